# baseline (speedup 1.0000x reference)
import functools
import math

import jax
import jax.numpy as jnp
from jax import lax
from jax.experimental import pallas as pl
from jax.experimental.pallas import tpu as pltpu
from jax.experimental.pallas import tpu_sc as plsc

N = 50000
E = 800000
SDIM = 64
VDIM = 16
EDIM = 16
NL = 5
CUTOFF = 5.0

NB_NODE = 5000
EB_EDGE = 3200

_WGT = lambda shp: pl.BlockSpec(shp, lambda i: (0,) * len(shp))


def _ln_proj_body(s_ref, v_ref, g_ref, b_ref, vg_ref, wd_ref, ws_ref,
                  sln_ref, vln_ref, td_ref, ts_ref):
    s = s_ref[...]
    mu = jnp.mean(s, axis=-1, keepdims=True)
    xc = s - mu
    var = jnp.mean(xc * xc, axis=-1, keepdims=True)
    sln = xc * jax.lax.rsqrt(var + 1e-6) * g_ref[...] + b_ref[...]
    v = v_ref[...]
    vn2 = jnp.sum(v * v, axis=-1, keepdims=True) * (1.0 / VDIM)
    vln = v * jax.lax.rsqrt(vn2 + 1e-6) * vg_ref[...]
    sln_ref[...] = sln
    vln_ref[...] = vln
    nb = s.shape[0]
    ad = jnp.dot(sln, wd_ref[...], preferred_element_type=jnp.float32)
    asr = jnp.dot(sln, ws_ref[...], preferred_element_type=jnp.float32)
    td_ref[...] = jnp.concatenate(
        [ad, jnp.zeros((nb, 128 - SDIM), jnp.float32)], axis=-1)
    ts_ref[...] = jnp.concatenate(
        [asr, vln, jnp.zeros((nb, 128 - SDIM - 3 * VDIM), jnp.float32)],
        axis=-1)


def _ln_proj(s, v48, g, b, vg48, wdst, wsrc):
    nb = pl.BlockSpec((NB_NODE, SDIM), lambda i: (i, 0))
    vb = pl.BlockSpec((NB_NODE, 3 * VDIM), lambda i: (i, 0))
    tb = pl.BlockSpec((NB_NODE, 128), lambda i: (i, 0))
    return pl.pallas_call(
        _ln_proj_body,
        grid=(N // NB_NODE,),
        in_specs=[nb, vb, _WGT((1, SDIM)), _WGT((1, SDIM)), _WGT((1, 3 * VDIM)),
                  _WGT((SDIM, SDIM)), _WGT((SDIM, SDIM))],
        out_specs=[nb, vb, tb, tb],
        out_shape=[
            jax.ShapeDtypeStruct((N, SDIM), jnp.float32),
            jax.ShapeDtypeStruct((N, 3 * VDIM), jnp.float32),
            jax.ShapeDtypeStruct((N, 128), jnp.float32),
            jax.ShapeDtypeStruct((N, 128), jnp.float32),
        ],
    )(s, v48, g.reshape(1, -1), b.reshape(1, -1), vg48.reshape(1, -1), wdst, wsrc)


def _ln_out_body(s_ref, v_ref, g_ref, b_ref, vg_ref, sln_ref, vln_ref):
    s = s_ref[...]
    mu = jnp.mean(s, axis=-1, keepdims=True)
    xc = s - mu
    var = jnp.mean(xc * xc, axis=-1, keepdims=True)
    sln_ref[...] = xc * jax.lax.rsqrt(var + 1e-6) * g_ref[...] + b_ref[...]
    v = v_ref[...]
    vn2 = jnp.sum(v * v, axis=-1, keepdims=True) * (1.0 / VDIM)
    vln_ref[...] = v * jax.lax.rsqrt(vn2 + 1e-6) * vg_ref[...]


def _ln_out(s, v48, g, b, vg48):
    nb = pl.BlockSpec((NB_NODE, SDIM), lambda i: (i, 0))
    vb = pl.BlockSpec((NB_NODE, 3 * VDIM), lambda i: (i, 0))
    return pl.pallas_call(
        _ln_out_body,
        grid=(N // NB_NODE,),
        in_specs=[nb, vb, _WGT((1, SDIM)), _WGT((1, SDIM)), _WGT((1, 3 * VDIM))],
        out_specs=[nb, vb],
        out_shape=[
            jax.ShapeDtypeStruct((N, SDIM), jnp.float32),
            jax.ShapeDtypeStruct((N, 3 * VDIM), jnp.float32),
        ],
    )(s, v48, g.reshape(1, -1), b.reshape(1, -1), vg48.reshape(1, -1))


def _edge_body(has_v, gd_ref, gs_ref, dae_ref, r_ref, w1c_ref, b1_ref,
               w2_ref, b2_ref, msg_ref):
    dae = dae_ref[...]
    gd = gd_ref[...]
    gs = gs_ref[...]
    x = (gd[:, :SDIM] + gs[:, :SDIM]
         + jnp.dot(dae, w1c_ref[...], preferred_element_type=jnp.float32)
         + b1_ref[...])
    h = x * jax.nn.sigmoid(x)
    m = jnp.dot(h, w2_ref[...], preferred_element_type=jnp.float32) + b2_ref[...]
    d = dae[:, 0:1]
    w = 0.5 * (jnp.cos((math.pi / CUTOFF) * d) + 1.0) * (d < CUTOFF).astype(jnp.float32)
    ms = m[:, :SDIM] * w
    gr = m[:, SDIM:SDIM + VDIM] * w
    gv = m[:, SDIM + VDIM:] * w
    r = r_ref[...]
    parts = [ms]
    for c in range(3):
        vc = gr * r[:, c:c + 1]
        if has_v:
            vc = vc + gv * gs[:, SDIM + c * VDIM:SDIM + (c + 1) * VDIM]
        parts.append(vc)
    ne = dae.shape[0]
    parts.append(jnp.ones((ne, 1), jnp.float32))
    parts.append(jnp.zeros((ne, 15), jnp.float32))
    msg_ref[...] = jnp.concatenate(parts, axis=-1)


def _edge_mlp(gd, gs, dae, r3, w1c, b1, w2, b2, has_v):
    ne = dae.shape[0]
    tb = pl.BlockSpec((EB_EDGE, 128), lambda i: (i, 0))
    ins = [tb, tb, pl.BlockSpec((EB_EDGE, 18), lambda i: (i, 0)),
           pl.BlockSpec((EB_EDGE, 3), lambda i: (i, 0)),
           _WGT((18, SDIM)), _WGT((1, SDIM)),
           _WGT((SDIM, SDIM + 2 * VDIM)), _WGT((1, SDIM + 2 * VDIM))]
    return pl.pallas_call(
        functools.partial(_edge_body, has_v),
        grid=(ne // EB_EDGE,),
        in_specs=ins,
        out_specs=tb,
        out_shape=jax.ShapeDtypeStruct((ne, 128), jnp.float32),
    )(gd, gs, dae, r3, w1c, b1.reshape(1, -1), w2, b2.reshape(1, -1))


def _update_body(has_mlp, sln_ref, vln_ref, agg_ref, w1a_ref, w1b_ref,
                 b1_ref, w2_ref, b2_ref, s_ref, v_ref):
    agg = agg_ref[...]
    sagg = agg[:, :SDIM]
    vsum = agg[:, SDIM:SDIM + 3 * VDIM]
    cnt = agg[:, SDIM + 3 * VDIM:SDIM + 3 * VDIM + 1]
    s_new = sln_ref[...] + sagg
    inv = 1.0 / jnp.maximum(cnt, 1.0)
    v_new = vln_ref[...] + vsum * inv
    if has_mlp:
        x = (jnp.dot(s_new, w1a_ref[...], preferred_element_type=jnp.float32)
             + jnp.dot(sagg, w1b_ref[...], preferred_element_type=jnp.float32)
             + b1_ref[...])
        h = x * jax.nn.sigmoid(x)
        u = jnp.dot(h, w2_ref[...], preferred_element_type=jnp.float32) + b2_ref[...]
        s_new = s_new + u[:, :SDIM]
        gate = jax.nn.sigmoid(u[:, SDIM:])
        v_new = v_new * jnp.concatenate([gate, gate, gate], axis=-1)
    s_ref[...] = s_new
    v_ref[...] = v_new


def _update(sln, vln, agg, w1a, w1b, b1, w2, b2, has_mlp):
    nb = pl.BlockSpec((NB_NODE, SDIM), lambda i: (i, 0))
    vb = pl.BlockSpec((NB_NODE, 3 * VDIM), lambda i: (i, 0))
    tb = pl.BlockSpec((NB_NODE, 128), lambda i: (i, 0))
    return pl.pallas_call(
        functools.partial(_update_body, has_mlp),
        grid=(N // NB_NODE,),
        in_specs=[nb, vb, tb, _WGT((SDIM, SDIM)), _WGT((SDIM, SDIM)),
                  _WGT((1, SDIM)), _WGT((SDIM, SDIM + VDIM)), _WGT((1, SDIM + VDIM))],
        out_specs=[nb, vb],
        out_shape=[
            jax.ShapeDtypeStruct((N, SDIM), jnp.float32),
            jax.ShapeDtypeStruct((N, 3 * VDIM), jnp.float32),
        ],
    )(sln, vln, agg, w1a, w1b, b1.reshape(1, -1), w2, b2.reshape(1, -1))


E_PAD = 819200
NWORK = 32
ROWS_PW = E_PAD // (NWORK * 128)
GCH = 2


def _sc_gather_call():
    mesh = plsc.VectorSubcoreMesh(core_axis_name="c", subcore_axis_name="s")
    outs = [jax.ShapeDtypeStruct((E_PAD, 128), jnp.float32),
            jax.ShapeDtypeStruct((E_PAD, 128), jnp.float32)]
    scratch = [
        pltpu.VMEM((GCH, 128), jnp.int32),
        pltpu.VMEM((GCH, 128), jnp.int32),
        pltpu.VMEM((GCH * 128, 128), jnp.float32),
        pltpu.VMEM((GCH * 128, 128), jnp.float32),
        pltpu.SemaphoreType.DMA,
        pltpu.SemaphoreType.DMA,
        pltpu.SemaphoreType.DMA,
    ]

    def body(td_hbm, ts_hbm, dst_hbm, src_hbm, gd_hbm, gs_hbm,
             idxd, idxs, bufD, bufS, semI, semG, semO):
        wid = lax.axis_index("s") * 2 + lax.axis_index("c")
        row0 = wid * ROWS_PW

        def chunk(i, carry):
            r = row0 + i * GCH
            e0 = r * 128
            ci1 = pltpu.async_copy(dst_hbm.at[pl.ds(r, GCH)], idxd, semI)
            ci2 = pltpu.async_copy(src_hbm.at[pl.ds(r, GCH)], idxs, semI)
            ci1.wait()
            ci2.wait()
            g = []
            for j in range(GCH):
                sl = pl.ds(j * 128, 128)
                g.append(pltpu.async_copy(td_hbm.at[idxd.at[j]], bufD.at[sl], semG))
                g.append(pltpu.async_copy(ts_hbm.at[idxs.at[j]], bufS.at[sl], semG))
            for c in g:
                c.wait()
            o = [pltpu.async_copy(bufD, gd_hbm.at[pl.ds(e0, GCH * 128)], semO),
                 pltpu.async_copy(bufS, gs_hbm.at[pl.ds(e0, GCH * 128)], semO)]
            for c in o:
                c.wait()
            return carry

        lax.fori_loop(0, ROWS_PW // GCH, chunk, 0)

    return pl.kernel(body, out_type=outs, mesh=mesh, scratch_types=scratch)


def _gather(td, ts, dst2d, src2d):
    return _sc_gather_call()(td, ts, dst2d, src2d)


PASS_NODES = 12248
ACC_ROWS = 12256
MINI = 1056
N_OUT = 50048
IDX_ROWS = E_PAD // 128
ROWS_PT = IDX_ROWS // 16
SG = 2


def _sc_scatter_call():
    mesh = plsc.VectorSubcoreMesh(core_axis_name="c", subcore_axis_name="s")
    out = jax.ShapeDtypeStruct((N_OUT, 128), jnp.float32)
    scratch = [
        pltpu.VMEM((SG, 128), jnp.int32),
        pltpu.VMEM((SG * 128, 128), jnp.float32),
        pltpu.VMEM_SHARED((ACC_ROWS, 128), jnp.float32),
        pltpu.SemaphoreType.DMA,
        pltpu.SemaphoreType.DMA,
        pltpu.SemaphoreType.DMA,
    ]

    def body(msg_hbm, lidx_hbm, zeros_hbm, agg_hbm, idxb, mbuf, acc,
             semI, semM, semO):
        cid = lax.axis_index("c")
        sid = lax.axis_index("s")
        row0 = sid * ROWS_PT

        def zero_acc():
            @pl.when(sid < 15)
            def _():
                pltpu.async_copy(zeros_hbm, acc.at[pl.ds(sid * 768, 768)],
                                 semO).wait()

            @pl.when(sid == 15)
            def _():
                pltpu.async_copy(zeros_hbm.at[pl.ds(0, 736)],
                                 acc.at[pl.ds(11520, 736)], semO).wait()

        def stream_scatter(pidx):
            def grp(i, carry):
                r = row0 + i * SG
                ci = pltpu.async_copy(
                    lidx_hbm.at[pl.ds(pidx * IDX_ROWS + r, SG)], idxb, semI)
                cm = pltpu.async_copy(
                    msg_hbm.at[pl.ds(r * 128, SG * 128)], mbuf, semM)
                ci.wait()
                cm.wait()
                ops = []
                for j in range(SG):
                    sl = pl.ds(j * 128, 128)
                    ops.append(pltpu.async_copy(mbuf.at[sl], acc.at[idxb.at[j]],
                                                semO, add=True))
                for o in ops:
                    o.wait()
                return carry

            lax.fori_loop(0, ROWS_PT // SG, grp, 0)

        for k_local in range(2):
            kidx = 2 * cid + k_local
            base = pl.multiple_of(MINI + kidx * PASS_NODES, 8)
            zero_acc()
            plsc.subcore_barrier()
            stream_scatter(kidx)
            plsc.subcore_barrier()

            @pl.when(sid < 15)
            def _():
                pltpu.async_copy(
                    acc.at[pl.ds(sid * 768, 768)],
                    agg_hbm.at[pl.ds(base + sid * 768, 768)], semO).wait()

            @pl.when(sid == 15)
            def _():
                pltpu.async_copy(
                    acc.at[pl.ds(11520, 728)],
                    agg_hbm.at[pl.ds(base + 11520, 728)], semO).wait()

            plsc.subcore_barrier()

        zero_acc()
        plsc.subcore_barrier()
        stream_scatter(4)
        plsc.subcore_barrier()

        @pl.when((cid == 0) & (sid < 8))
        def _():
            pltpu.async_copy(acc.at[pl.ds(sid * 128, 128)],
                             agg_hbm.at[pl.ds(sid * 128, 128)], semO).wait()

        @pl.when((cid == 0) & (sid == 8))
        def _():
            pltpu.async_copy(acc.at[pl.ds(1024, 32)],
                             agg_hbm.at[pl.ds(1024, 32)], semO).wait()

    return pl.kernel(body, out_type=out, mesh=mesh, scratch_types=scratch)


def _scatter(msg, lidx2d, zeros784):
    return _sc_scatter_call()(msg, lidx2d, zeros784)


def kernel(s, v, p, edge_index_local, d_local, a_local, r_local, e_local,
           edge_index_global, d_global, a_global, r_global, e_global, batch, params):
    src = edge_index_local[0]
    dst = edge_index_local[1]
    v48 = v.reshape(N, 3 * VDIM)
    npad = E_PAD - E
    pad_ids = (jnp.arange(npad, dtype=jnp.int32) * 997) % N
    dst2d = jnp.concatenate([dst, pad_ids]).reshape(E_PAD // 128, 128)
    src2d = jnp.concatenate([src, pad_ids]).reshape(E_PAD // 128, 128)
    dst_seg = jnp.concatenate([dst, jnp.full((npad,), N, jnp.int32)])
    dae = jnp.concatenate([d_local[:, None], a_local[:, None], e_local], axis=-1)
    dae = jnp.concatenate([dae, jnp.zeros((npad, 18), jnp.float32)], axis=0)
    r_pad = jnp.concatenate([r_local, jnp.zeros((npad, 3), jnp.float32)], axis=0)

    spread8 = jnp.arange(E_PAD, dtype=jnp.int32) % 8
    dump = PASS_NODES + spread8
    lidx_parts = []
    for k in range(4):
        base = MINI + k * PASS_NODES
        inr = (dst_seg >= base) & (dst_seg < base + PASS_NODES)
        lidx_parts.append(jnp.where(inr, dst_seg - base, dump))
    lidx_parts.append(jnp.where(dst_seg < MINI, dst_seg, MINI + spread8))
    lidx2d = jnp.concatenate(lidx_parts).reshape(5 * IDX_ROWS, 128)
    zeros784 = jnp.zeros((768, 128), jnp.float32)

    for i in range(NL):
        lp = params["layers"][i]
        has_v = i > 0
        has_mlp = i < NL - 1
        vg48 = jnp.tile(lp["ln_vg"], 3)
        w1_dst = lp["eW1"][:SDIM]
        w1_src = lp["eW1"][SDIM:2 * SDIM]
        w1_c = lp["eW1"][2 * SDIM:]
        sln, vln, td, ts = _ln_proj(s, v48, lp["ln_g"], lp["ln_b"], vg48,
                                    w1_dst, w1_src)
        gd, gs = _gather(td, ts, dst2d, src2d)
        msg = _edge_mlp(gd, gs, dae, r_pad, w1_c, lp["eb1"],
                        lp["eW2"], lp["eb2"], has_v)
        agg = _scatter(msg, lidx2d, zeros784)
        s, v48 = _update(sln, vln, agg,
                         lp["uW1"][:SDIM], lp["uW1"][SDIM:], lp["ub1"],
                         lp["uW2"], lp["ub2"], has_mlp)

    on = params["out_norm"]
    s, v48 = _ln_out(s, v48, on["g"], on["b"], jnp.tile(on["vg"], 3))
    return (s, v48.reshape(N, 3, VDIM))


# speedup vs baseline: 1.0724x; 1.0724x over previous
import functools
import math

import jax
import jax.numpy as jnp
from jax import lax
from jax.experimental import pallas as pl
from jax.experimental.pallas import tpu as pltpu
from jax.experimental.pallas import tpu_sc as plsc

N = 50000
E = 800000
SDIM = 64
VDIM = 16
EDIM = 16
NL = 5
CUTOFF = 5.0

NB_NODE = 5000
EB_EDGE = 3200

_WGT = lambda shp: pl.BlockSpec(shp, lambda i: (0,) * len(shp))


def _ln_proj_body(s_ref, v_ref, g_ref, b_ref, vg_ref, wd_ref, ws_ref,
                  sln_ref, vln_ref, td_ref, ts_ref):
    s = s_ref[...]
    mu = jnp.mean(s, axis=-1, keepdims=True)
    xc = s - mu
    var = jnp.mean(xc * xc, axis=-1, keepdims=True)
    sln = xc * jax.lax.rsqrt(var + 1e-6) * g_ref[...] + b_ref[...]
    v = v_ref[...]
    vn2 = jnp.sum(v * v, axis=-1, keepdims=True) * (1.0 / VDIM)
    vln = v * jax.lax.rsqrt(vn2 + 1e-6) * vg_ref[...]
    sln_ref[...] = sln
    vln_ref[...] = vln
    nb = s.shape[0]
    ad = jnp.dot(sln, wd_ref[...], preferred_element_type=jnp.float32)
    asr = jnp.dot(sln, ws_ref[...], preferred_element_type=jnp.float32)
    td_ref[...] = jnp.concatenate(
        [ad, jnp.zeros((nb, 128 - SDIM), jnp.float32)], axis=-1)
    ts_ref[...] = jnp.concatenate(
        [asr, vln, jnp.zeros((nb, 128 - SDIM - 3 * VDIM), jnp.float32)],
        axis=-1)


def _ln_proj(s, v48, g, b, vg48, wdst, wsrc):
    nb = pl.BlockSpec((NB_NODE, SDIM), lambda i: (i, 0))
    vb = pl.BlockSpec((NB_NODE, 3 * VDIM), lambda i: (i, 0))
    tb = pl.BlockSpec((NB_NODE, 128), lambda i: (i, 0))
    return pl.pallas_call(
        _ln_proj_body,
        grid=(N // NB_NODE,),
        in_specs=[nb, vb, _WGT((1, SDIM)), _WGT((1, SDIM)), _WGT((1, 3 * VDIM)),
                  _WGT((SDIM, SDIM)), _WGT((SDIM, SDIM))],
        out_specs=[nb, vb, tb, tb],
        out_shape=[
            jax.ShapeDtypeStruct((N, SDIM), jnp.float32),
            jax.ShapeDtypeStruct((N, 3 * VDIM), jnp.float32),
            jax.ShapeDtypeStruct((N, 128), jnp.float32),
            jax.ShapeDtypeStruct((N, 128), jnp.float32),
        ],
    )(s, v48, g.reshape(1, -1), b.reshape(1, -1), vg48.reshape(1, -1), wdst, wsrc)


def _ln_out_body(s_ref, v_ref, g_ref, b_ref, vg_ref, sln_ref, vln_ref):
    s = s_ref[...]
    mu = jnp.mean(s, axis=-1, keepdims=True)
    xc = s - mu
    var = jnp.mean(xc * xc, axis=-1, keepdims=True)
    sln_ref[...] = xc * jax.lax.rsqrt(var + 1e-6) * g_ref[...] + b_ref[...]
    v = v_ref[...]
    vn2 = jnp.sum(v * v, axis=-1, keepdims=True) * (1.0 / VDIM)
    vln_ref[...] = v * jax.lax.rsqrt(vn2 + 1e-6) * vg_ref[...]


def _ln_out(s, v48, g, b, vg48):
    nb = pl.BlockSpec((NB_NODE, SDIM), lambda i: (i, 0))
    vb = pl.BlockSpec((NB_NODE, 3 * VDIM), lambda i: (i, 0))
    return pl.pallas_call(
        _ln_out_body,
        grid=(N // NB_NODE,),
        in_specs=[nb, vb, _WGT((1, SDIM)), _WGT((1, SDIM)), _WGT((1, 3 * VDIM))],
        out_specs=[nb, vb],
        out_shape=[
            jax.ShapeDtypeStruct((N, SDIM), jnp.float32),
            jax.ShapeDtypeStruct((N, 3 * VDIM), jnp.float32),
        ],
    )(s, v48, g.reshape(1, -1), b.reshape(1, -1), vg48.reshape(1, -1))


def _edge_body(has_v, gd_ref, gs_ref, dae_ref, r_ref, w1c_ref, b1_ref,
               w2_ref, b2_ref, msg_ref):
    dae = dae_ref[...]
    gd = gd_ref[...]
    gs = gs_ref[...]
    x = (gd[:, :SDIM] + gs[:, :SDIM]
         + jnp.dot(dae, w1c_ref[...], preferred_element_type=jnp.float32)
         + b1_ref[...])
    h = x * jax.nn.sigmoid(x)
    m = jnp.dot(h, w2_ref[...], preferred_element_type=jnp.float32) + b2_ref[...]
    d = dae[:, 0:1]
    w = 0.5 * (jnp.cos((math.pi / CUTOFF) * d) + 1.0) * (d < CUTOFF).astype(jnp.float32)
    ms = m[:, :SDIM] * w
    gr = m[:, SDIM:SDIM + VDIM] * w
    gv = m[:, SDIM + VDIM:] * w
    r = r_ref[...]
    parts = [ms]
    for c in range(3):
        vc = gr * r[:, c:c + 1]
        if has_v:
            vc = vc + gv * gs[:, SDIM + c * VDIM:SDIM + (c + 1) * VDIM]
        parts.append(vc)
    ne = dae.shape[0]
    parts.append(jnp.ones((ne, 1), jnp.float32))
    parts.append(jnp.zeros((ne, 15), jnp.float32))
    msg_ref[...] = jnp.concatenate(parts, axis=-1)


def _edge_mlp(gd, gs, dae, r3, w1c, b1, w2, b2, has_v):
    ne = dae.shape[0]
    tb = pl.BlockSpec((EB_EDGE, 128), lambda i: (i, 0))
    ins = [tb, tb, pl.BlockSpec((EB_EDGE, 18), lambda i: (i, 0)),
           pl.BlockSpec((EB_EDGE, 3), lambda i: (i, 0)),
           _WGT((18, SDIM)), _WGT((1, SDIM)),
           _WGT((SDIM, SDIM + 2 * VDIM)), _WGT((1, SDIM + 2 * VDIM))]
    return pl.pallas_call(
        functools.partial(_edge_body, has_v),
        grid=(ne // EB_EDGE,),
        in_specs=ins,
        out_specs=tb,
        out_shape=jax.ShapeDtypeStruct((ne, 128), jnp.float32),
    )(gd, gs, dae, r3, w1c, b1.reshape(1, -1), w2, b2.reshape(1, -1))


def _update_body(has_mlp, sln_ref, vln_ref, agg_ref, w1a_ref, w1b_ref,
                 b1_ref, w2_ref, b2_ref, s_ref, v_ref):
    agg = agg_ref[...]
    sagg = agg[:, :SDIM]
    vsum = agg[:, SDIM:SDIM + 3 * VDIM]
    cnt = agg[:, SDIM + 3 * VDIM:SDIM + 3 * VDIM + 1]
    s_new = sln_ref[...] + sagg
    inv = 1.0 / jnp.maximum(cnt, 1.0)
    v_new = vln_ref[...] + vsum * inv
    if has_mlp:
        x = (jnp.dot(s_new, w1a_ref[...], preferred_element_type=jnp.float32)
             + jnp.dot(sagg, w1b_ref[...], preferred_element_type=jnp.float32)
             + b1_ref[...])
        h = x * jax.nn.sigmoid(x)
        u = jnp.dot(h, w2_ref[...], preferred_element_type=jnp.float32) + b2_ref[...]
        s_new = s_new + u[:, :SDIM]
        gate = jax.nn.sigmoid(u[:, SDIM:])
        v_new = v_new * jnp.concatenate([gate, gate, gate], axis=-1)
    s_ref[...] = s_new
    v_ref[...] = v_new


def _update(sln, vln, agg, w1a, w1b, b1, w2, b2, has_mlp):
    nb = pl.BlockSpec((NB_NODE, SDIM), lambda i: (i, 0))
    vb = pl.BlockSpec((NB_NODE, 3 * VDIM), lambda i: (i, 0))
    tb = pl.BlockSpec((NB_NODE, 128), lambda i: (i, 0))
    return pl.pallas_call(
        functools.partial(_update_body, has_mlp),
        grid=(N // NB_NODE,),
        in_specs=[nb, vb, tb, _WGT((SDIM, SDIM)), _WGT((SDIM, SDIM)),
                  _WGT((1, SDIM)), _WGT((SDIM, SDIM + VDIM)), _WGT((1, SDIM + VDIM))],
        out_specs=[nb, vb],
        out_shape=[
            jax.ShapeDtypeStruct((N, SDIM), jnp.float32),
            jax.ShapeDtypeStruct((N, 3 * VDIM), jnp.float32),
        ],
    )(sln, vln, agg, w1a, w1b, b1.reshape(1, -1), w2, b2.reshape(1, -1))


E_PAD = 819200
NWORK = 32
ROWS_PW = E_PAD // (NWORK * 128)


def _sc_gather_call():
    mesh = plsc.VectorSubcoreMesh(core_axis_name="c", subcore_axis_name="s")
    outs = [jax.ShapeDtypeStruct((E_PAD, 128), jnp.float32),
            jax.ShapeDtypeStruct((E_PAD, 128), jnp.float32)]
    scratch = [
        pltpu.VMEM((1, 128), jnp.int32),
        pltpu.VMEM((1, 128), jnp.int32),
        pltpu.VMEM((1, 128), jnp.int32),
        pltpu.VMEM((1, 128), jnp.int32),
        pltpu.VMEM((128, 128), jnp.float32),
        pltpu.VMEM((128, 128), jnp.float32),
        pltpu.VMEM((128, 128), jnp.float32),
        pltpu.VMEM((128, 128), jnp.float32),
        pltpu.SemaphoreType.DMA,
        pltpu.SemaphoreType.DMA,
        pltpu.SemaphoreType.DMA,
        pltpu.SemaphoreType.DMA,
        pltpu.SemaphoreType.DMA,
    ]

    def body(td_hbm, ts_hbm, dst_hbm, src_hbm, gd_hbm, gs_hbm,
             idxd0, idxd1, idxs0, idxs1, bufD0, bufD1, bufS0, bufS1,
             semI0, semI1, semG, semW0, semW1):
        wid = lax.axis_index("s") * 2 + lax.axis_index("c")
        row0 = wid * ROWS_PW
        idxd = (idxd0, idxd1)
        idxs = (idxs0, idxs1)
        bufD = (bufD0, bufD1)
        bufS = (bufS0, bufS1)
        semI = (semI0, semI1)
        semW = (semW0, semW1)
        nit = ROWS_PW

        def issue_idx(it, b):
            r = row0 + it
            pltpu.async_copy(dst_hbm.at[pl.ds(r, 1)], idxd[b], semI[b])
            pltpu.async_copy(src_hbm.at[pl.ds(r, 1)], idxs[b], semI[b])

        def wait_idx(b):
            pltpu.make_async_copy(dst_hbm.at[pl.ds(0, 1)], idxd[b],
                                  semI[b]).wait()
            pltpu.make_async_copy(src_hbm.at[pl.ds(0, 1)], idxs[b],
                                  semI[b]).wait()

        def wait_wb(b):
            pltpu.make_async_copy(bufD[b], gd_hbm.at[pl.ds(0, 128)],
                                  semW[b]).wait()
            pltpu.make_async_copy(bufS[b], gs_hbm.at[pl.ds(0, 128)],
                                  semW[b]).wait()

        issue_idx(0, 0)

        def pair(p, carry):
            for b in range(2):
                it = p * 2 + b
                wait_idx(b)

                @pl.when(it + 1 < nit)
                def _():
                    issue_idx(it + 1, 1 - b)

                @pl.when(it >= 2)
                def _():
                    wait_wb(b)

                g = [pltpu.async_copy(td_hbm.at[idxd[b].at[0]], bufD[b], semG),
                     pltpu.async_copy(ts_hbm.at[idxs[b].at[0]], bufS[b], semG)]
                for c in g:
                    c.wait()
                e0 = (row0 + it) * 128
                pltpu.async_copy(bufD[b], gd_hbm.at[pl.ds(e0, 128)], semW[b])
                pltpu.async_copy(bufS[b], gs_hbm.at[pl.ds(e0, 128)], semW[b])
            return carry

        lax.fori_loop(0, nit // 2, pair, 0)
        wait_wb(0)
        wait_wb(1)

    return pl.kernel(body, out_type=outs, mesh=mesh, scratch_types=scratch)


def _gather(td, ts, dst2d, src2d):
    return _sc_gather_call()(td, ts, dst2d, src2d)


PASS_NODES = 12224
ACC_ROWS = 12232
MINI = 1152
N_OUT = 50048
IDX_ROWS = E_PAD // 128
ROWS_PT = IDX_ROWS // 16
SG = 1


def _sc_scatter_call():
    mesh = plsc.VectorSubcoreMesh(core_axis_name="c", subcore_axis_name="s")
    out = jax.ShapeDtypeStruct((N_OUT, 128), jnp.float32)
    scratch = [
        pltpu.VMEM((SG, 128), jnp.int32),
        pltpu.VMEM((SG, 128), jnp.int32),
        pltpu.VMEM((SG * 128, 128), jnp.float32),
        pltpu.VMEM((SG * 128, 128), jnp.float32),
        pltpu.VMEM_SHARED((ACC_ROWS, 128), jnp.float32),
        pltpu.SemaphoreType.DMA,
        pltpu.SemaphoreType.DMA,
        pltpu.SemaphoreType.DMA,
        pltpu.SemaphoreType.DMA,
        pltpu.SemaphoreType.DMA,
    ]

    def body(msg_hbm, lidx_hbm, zeros_hbm, agg_hbm, idxb0, idxb1, mbuf0,
             mbuf1, acc, semI0, semI1, semM0, semM1, semO):
        cid = lax.axis_index("c")
        sid = lax.axis_index("s")
        row0 = sid * ROWS_PT
        idxb = (idxb0, idxb1)
        mbuf = (mbuf0, mbuf1)
        semI = (semI0, semI1)
        semM = (semM0, semM1)

        def zero_acc():
            @pl.when(sid < 15)
            def _():
                pltpu.async_copy(zeros_hbm, acc.at[pl.ds(sid * 768, 768)],
                                 semO).wait()

            @pl.when(sid == 15)
            def _():
                pltpu.async_copy(zeros_hbm.at[pl.ds(0, 712)],
                                 acc.at[pl.ds(11520, 712)], semO).wait()

        def stream_scatter(pidx):
            nit = ROWS_PT // SG

            def issue_loads(it, b):
                r = row0 + it * SG
                pltpu.async_copy(
                    lidx_hbm.at[pl.ds(pidx * IDX_ROWS + r, SG)], idxb[b],
                    semI[b])
                pltpu.async_copy(
                    msg_hbm.at[pl.ds(r * 128, SG * 128)], mbuf[b], semM[b])

            def wait_loads(b):
                pltpu.make_async_copy(
                    lidx_hbm.at[pl.ds(0, SG)], idxb[b], semI[b]).wait()
                pltpu.make_async_copy(
                    msg_hbm.at[pl.ds(0, SG * 128)], mbuf[b], semM[b]).wait()

            issue_loads(0, 0)

            def pair(p, carry):
                for b in range(2):
                    it = p * 2 + b
                    wait_loads(b)

                    @pl.when(it + 1 < nit)
                    def _():
                        issue_loads(it + 1, 1 - b)

                    ops = []
                    for j in range(SG):
                        sl = pl.ds(j * 128, 128)
                        ops.append(pltpu.async_copy(
                            mbuf[b].at[sl], acc.at[idxb[b].at[j]], semO,
                            add=True))
                    for o in ops:
                        o.wait()
                return carry

            lax.fori_loop(0, nit // 2, pair, 0)

        for k_local in range(2):
            kidx = 2 * cid + k_local
            base = pl.multiple_of(MINI + kidx * PASS_NODES, 8)
            zero_acc()
            plsc.subcore_barrier()
            stream_scatter(kidx)
            plsc.subcore_barrier()

            @pl.when(sid < 15)
            def _():
                pltpu.async_copy(
                    acc.at[pl.ds(sid * 768, 768)],
                    agg_hbm.at[pl.ds(base + sid * 768, 768)], semO).wait()

            @pl.when(sid == 15)
            def _():
                pltpu.async_copy(
                    acc.at[pl.ds(11520, 704)],
                    agg_hbm.at[pl.ds(base + 11520, 704)], semO).wait()

            plsc.subcore_barrier()

        zero_acc()
        plsc.subcore_barrier()
        stream_scatter(4)
        plsc.subcore_barrier()

        @pl.when((cid == 0) & (sid < 9))
        def _():
            pltpu.async_copy(acc.at[pl.ds(sid * 128, 128)],
                             agg_hbm.at[pl.ds(sid * 128, 128)], semO).wait()

    return pl.kernel(body, out_type=out, mesh=mesh, scratch_types=scratch)


def _scatter(msg, lidx2d, zeros784):
    return _sc_scatter_call()(msg, lidx2d, zeros784)


def kernel(s, v, p, edge_index_local, d_local, a_local, r_local, e_local,
           edge_index_global, d_global, a_global, r_global, e_global, batch, params):
    src = edge_index_local[0]
    dst = edge_index_local[1]
    v48 = v.reshape(N, 3 * VDIM)
    npad = E_PAD - E
    pad_ids = (jnp.arange(npad, dtype=jnp.int32) * 997) % N
    dst2d = jnp.concatenate([dst, pad_ids]).reshape(E_PAD // 128, 128)
    src2d = jnp.concatenate([src, pad_ids]).reshape(E_PAD // 128, 128)
    dst_seg = jnp.concatenate([dst, jnp.full((npad,), N, jnp.int32)])
    dae = jnp.concatenate([d_local[:, None], a_local[:, None], e_local], axis=-1)
    dae = jnp.concatenate([dae, jnp.zeros((npad, 18), jnp.float32)], axis=0)
    r_pad = jnp.concatenate([r_local, jnp.zeros((npad, 3), jnp.float32)], axis=0)

    spread8 = jnp.arange(E_PAD, dtype=jnp.int32) % 8
    dump = PASS_NODES + spread8
    lidx_parts = []
    for k in range(4):
        base = MINI + k * PASS_NODES
        inr = (dst_seg >= base) & (dst_seg < base + PASS_NODES)
        lidx_parts.append(jnp.where(inr, dst_seg - base, dump))
    lidx_parts.append(jnp.where(dst_seg < MINI, dst_seg, MINI + spread8))
    lidx2d = jnp.concatenate(lidx_parts).reshape(5 * IDX_ROWS, 128)
    zeros784 = jnp.zeros((768, 128), jnp.float32)

    for i in range(NL):
        lp = params["layers"][i]
        has_v = i > 0
        has_mlp = i < NL - 1
        vg48 = jnp.tile(lp["ln_vg"], 3)
        w1_dst = lp["eW1"][:SDIM]
        w1_src = lp["eW1"][SDIM:2 * SDIM]
        w1_c = lp["eW1"][2 * SDIM:]
        sln, vln, td, ts = _ln_proj(s, v48, lp["ln_g"], lp["ln_b"], vg48,
                                    w1_dst, w1_src)
        gd, gs = _gather(td, ts, dst2d, src2d)
        msg = _edge_mlp(gd, gs, dae, r_pad, w1_c, lp["eb1"],
                        lp["eW2"], lp["eb2"], has_v)
        agg = _scatter(msg, lidx2d, zeros784)
        s, v48 = _update(sln, vln, agg,
                         lp["uW1"][:SDIM], lp["uW1"][SDIM:], lp["ub1"],
                         lp["uW2"], lp["ub2"], has_mlp)

    on = params["out_norm"]
    s, v48 = _ln_out(s, v48, on["g"], on["b"], jnp.tile(on["vg"], 3))
    return (s, v48.reshape(N, 3, VDIM))


# speedup vs baseline: 1.2976x; 1.2100x over previous
import functools
import math

import jax
import jax.numpy as jnp
from jax import lax
from jax.experimental import pallas as pl
from jax.experimental.pallas import tpu as pltpu
from jax.experimental.pallas import tpu_sc as plsc

N = 50000
E = 800000
SDIM = 64
VDIM = 16
EDIM = 16
NL = 5
CUTOFF = 5.0

NB_NODE = 5000
EB_EDGE = 3200

_WGT = lambda shp: pl.BlockSpec(shp, lambda i: (0,) * len(shp))


def _ln_proj_body(s_ref, v_ref, g_ref, b_ref, vg_ref, wd_ref, ws_ref,
                  sln_ref, vln_ref, td_ref, ts_ref):
    s = s_ref[...]
    mu = jnp.mean(s, axis=-1, keepdims=True)
    xc = s - mu
    var = jnp.mean(xc * xc, axis=-1, keepdims=True)
    sln = xc * jax.lax.rsqrt(var + 1e-6) * g_ref[...] + b_ref[...]
    v = v_ref[...]
    vn2 = jnp.sum(v * v, axis=-1, keepdims=True) * (1.0 / VDIM)
    vln = v * jax.lax.rsqrt(vn2 + 1e-6) * vg_ref[...]
    sln_ref[...] = sln
    vln_ref[...] = vln
    nb = s.shape[0]
    ad = jnp.dot(sln, wd_ref[...], preferred_element_type=jnp.float32)
    asr = jnp.dot(sln, ws_ref[...], preferred_element_type=jnp.float32)
    td_ref[...] = jnp.concatenate(
        [ad, jnp.zeros((nb, 128 - SDIM), jnp.float32)], axis=-1)
    ts_ref[...] = jnp.concatenate(
        [asr, vln, jnp.zeros((nb, 128 - SDIM - 3 * VDIM), jnp.float32)],
        axis=-1)


def _ln_proj(s, v48, g, b, vg48, wdst, wsrc):
    nb = pl.BlockSpec((NB_NODE, SDIM), lambda i: (i, 0))
    vb = pl.BlockSpec((NB_NODE, 3 * VDIM), lambda i: (i, 0))
    tb = pl.BlockSpec((NB_NODE, 128), lambda i: (i, 0))
    return pl.pallas_call(
        _ln_proj_body,
        grid=(N // NB_NODE,),
        in_specs=[nb, vb, _WGT((1, SDIM)), _WGT((1, SDIM)), _WGT((1, 3 * VDIM)),
                  _WGT((SDIM, SDIM)), _WGT((SDIM, SDIM))],
        out_specs=[nb, vb, tb, tb],
        out_shape=[
            jax.ShapeDtypeStruct((N, SDIM), jnp.float32),
            jax.ShapeDtypeStruct((N, 3 * VDIM), jnp.float32),
            jax.ShapeDtypeStruct((N, 128), jnp.float32),
            jax.ShapeDtypeStruct((N, 128), jnp.float32),
        ],
    )(s, v48, g.reshape(1, -1), b.reshape(1, -1), vg48.reshape(1, -1), wdst, wsrc)


def _ln_out_body(s_ref, v_ref, g_ref, b_ref, vg_ref, sln_ref, vln_ref):
    s = s_ref[...]
    mu = jnp.mean(s, axis=-1, keepdims=True)
    xc = s - mu
    var = jnp.mean(xc * xc, axis=-1, keepdims=True)
    sln_ref[...] = xc * jax.lax.rsqrt(var + 1e-6) * g_ref[...] + b_ref[...]
    v = v_ref[...]
    vn2 = jnp.sum(v * v, axis=-1, keepdims=True) * (1.0 / VDIM)
    vln_ref[...] = v * jax.lax.rsqrt(vn2 + 1e-6) * vg_ref[...]


def _ln_out(s, v48, g, b, vg48):
    nb = pl.BlockSpec((NB_NODE, SDIM), lambda i: (i, 0))
    vb = pl.BlockSpec((NB_NODE, 3 * VDIM), lambda i: (i, 0))
    return pl.pallas_call(
        _ln_out_body,
        grid=(N // NB_NODE,),
        in_specs=[nb, vb, _WGT((1, SDIM)), _WGT((1, SDIM)), _WGT((1, 3 * VDIM))],
        out_specs=[nb, vb],
        out_shape=[
            jax.ShapeDtypeStruct((N, SDIM), jnp.float32),
            jax.ShapeDtypeStruct((N, 3 * VDIM), jnp.float32),
        ],
    )(s, v48, g.reshape(1, -1), b.reshape(1, -1), vg48.reshape(1, -1))


def _edge_body(has_v, gd_ref, gs_ref, dae_ref, r_ref, w1c_ref, b1_ref,
               w2_ref, b2_ref, msg_ref):
    dae = dae_ref[...]
    gd = gd_ref[...]
    gs = gs_ref[...]
    x = (gd[:, :SDIM] + gs[:, :SDIM]
         + jnp.dot(dae, w1c_ref[...], preferred_element_type=jnp.float32)
         + b1_ref[...])
    h = x * jax.nn.sigmoid(x)
    m = jnp.dot(h, w2_ref[...], preferred_element_type=jnp.float32) + b2_ref[...]
    d = dae[:, 0:1]
    w = 0.5 * (jnp.cos((math.pi / CUTOFF) * d) + 1.0) * (d < CUTOFF).astype(jnp.float32)
    ms = m[:, :SDIM] * w
    gr = m[:, SDIM:SDIM + VDIM] * w
    gv = m[:, SDIM + VDIM:] * w
    r = r_ref[...]
    parts = [ms]
    for c in range(3):
        vc = gr * r[:, c:c + 1]
        if has_v:
            vc = vc + gv * gs[:, SDIM + c * VDIM:SDIM + (c + 1) * VDIM]
        parts.append(vc)
    ne = dae.shape[0]
    parts.append(jnp.ones((ne, 1), jnp.float32))
    parts.append(jnp.zeros((ne, 15), jnp.float32))
    msg_ref[...] = jnp.concatenate(parts, axis=-1)


def _edge_mlp(gd, gs, dae, r3, w1c, b1, w2, b2, has_v):
    ne = dae.shape[0]
    tb = pl.BlockSpec((EB_EDGE, 128), lambda i: (i, 0))
    ins = [tb, tb, pl.BlockSpec((EB_EDGE, 18), lambda i: (i, 0)),
           pl.BlockSpec((EB_EDGE, 3), lambda i: (i, 0)),
           _WGT((18, SDIM)), _WGT((1, SDIM)),
           _WGT((SDIM, SDIM + 2 * VDIM)), _WGT((1, SDIM + 2 * VDIM))]
    return pl.pallas_call(
        functools.partial(_edge_body, has_v),
        grid=(ne // EB_EDGE,),
        in_specs=ins,
        out_specs=tb,
        out_shape=jax.ShapeDtypeStruct((ne, 128), jnp.float32),
    )(gd, gs, dae, r3, w1c, b1.reshape(1, -1), w2, b2.reshape(1, -1))


def _update_body(has_mlp, sln_ref, vln_ref, agg_ref, w1a_ref, w1b_ref,
                 b1_ref, w2_ref, b2_ref, s_ref, v_ref):
    agg = agg_ref[...]
    sagg = agg[:, :SDIM]
    vsum = agg[:, SDIM:SDIM + 3 * VDIM]
    cnt = agg[:, SDIM + 3 * VDIM:SDIM + 3 * VDIM + 1]
    s_new = sln_ref[...] + sagg
    inv = 1.0 / jnp.maximum(cnt, 1.0)
    v_new = vln_ref[...] + vsum * inv
    if has_mlp:
        x = (jnp.dot(s_new, w1a_ref[...], preferred_element_type=jnp.float32)
             + jnp.dot(sagg, w1b_ref[...], preferred_element_type=jnp.float32)
             + b1_ref[...])
        h = x * jax.nn.sigmoid(x)
        u = jnp.dot(h, w2_ref[...], preferred_element_type=jnp.float32) + b2_ref[...]
        s_new = s_new + u[:, :SDIM]
        gate = jax.nn.sigmoid(u[:, SDIM:])
        v_new = v_new * jnp.concatenate([gate, gate, gate], axis=-1)
    s_ref[...] = s_new
    v_ref[...] = v_new


def _update(sln, vln, agg, w1a, w1b, b1, w2, b2, has_mlp):
    nb = pl.BlockSpec((NB_NODE, SDIM), lambda i: (i, 0))
    vb = pl.BlockSpec((NB_NODE, 3 * VDIM), lambda i: (i, 0))
    tb = pl.BlockSpec((NB_NODE, 128), lambda i: (i, 0))
    return pl.pallas_call(
        functools.partial(_update_body, has_mlp),
        grid=(N // NB_NODE,),
        in_specs=[nb, vb, tb, _WGT((SDIM, SDIM)), _WGT((SDIM, SDIM)),
                  _WGT((1, SDIM)), _WGT((SDIM, SDIM + VDIM)), _WGT((1, SDIM + VDIM))],
        out_specs=[nb, vb],
        out_shape=[
            jax.ShapeDtypeStruct((N, SDIM), jnp.float32),
            jax.ShapeDtypeStruct((N, 3 * VDIM), jnp.float32),
        ],
    )(sln, vln, agg, w1a, w1b, b1.reshape(1, -1), w2, b2.reshape(1, -1))


E_PAD = 819200
NWORK = 32
ROWS_PW = E_PAD // (NWORK * 128)


def _sc_gather_call():
    mesh = plsc.VectorSubcoreMesh(core_axis_name="c", subcore_axis_name="s")
    outs = [jax.ShapeDtypeStruct((E_PAD, 128), jnp.float32),
            jax.ShapeDtypeStruct((E_PAD, 128), jnp.float32)]
    scratch = [
        pltpu.VMEM((1, 128), jnp.int32),
        pltpu.VMEM((1, 128), jnp.int32),
        pltpu.VMEM((1, 128), jnp.int32),
        pltpu.VMEM((1, 128), jnp.int32),
        pltpu.VMEM((128, 128), jnp.float32),
        pltpu.VMEM((128, 128), jnp.float32),
        pltpu.VMEM((128, 128), jnp.float32),
        pltpu.VMEM((128, 128), jnp.float32),
        pltpu.SemaphoreType.DMA,
        pltpu.SemaphoreType.DMA,
        pltpu.SemaphoreType.DMA,
        pltpu.SemaphoreType.DMA,
        pltpu.SemaphoreType.DMA,
    ]

    def body(td_hbm, ts_hbm, dst_hbm, src_hbm, gd_hbm, gs_hbm,
             idxd0, idxd1, idxs0, idxs1, bufD0, bufD1, bufS0, bufS1,
             semI0, semI1, semG, semW0, semW1):
        wid = lax.axis_index("s") * 2 + lax.axis_index("c")
        row0 = wid * ROWS_PW
        idxd = (idxd0, idxd1)
        idxs = (idxs0, idxs1)
        bufD = (bufD0, bufD1)
        bufS = (bufS0, bufS1)
        semI = (semI0, semI1)
        semW = (semW0, semW1)
        nit = ROWS_PW

        def issue_idx(it, b):
            r = row0 + it
            pltpu.async_copy(dst_hbm.at[pl.ds(r, 1)], idxd[b], semI[b])
            pltpu.async_copy(src_hbm.at[pl.ds(r, 1)], idxs[b], semI[b])

        def wait_idx(b):
            pltpu.make_async_copy(dst_hbm.at[pl.ds(0, 1)], idxd[b],
                                  semI[b]).wait()
            pltpu.make_async_copy(src_hbm.at[pl.ds(0, 1)], idxs[b],
                                  semI[b]).wait()

        def wait_wb(b):
            pltpu.make_async_copy(bufD[b], gd_hbm.at[pl.ds(0, 128)],
                                  semW[b]).wait()
            pltpu.make_async_copy(bufS[b], gs_hbm.at[pl.ds(0, 128)],
                                  semW[b]).wait()

        issue_idx(0, 0)

        def pair(p, carry):
            for b in range(2):
                it = p * 2 + b
                wait_idx(b)

                @pl.when(it + 1 < nit)
                def _():
                    issue_idx(it + 1, 1 - b)

                @pl.when(it >= 2)
                def _():
                    wait_wb(b)

                g = [pltpu.async_copy(td_hbm.at[idxd[b].at[0]], bufD[b], semG),
                     pltpu.async_copy(ts_hbm.at[idxs[b].at[0]], bufS[b], semG)]
                for c in g:
                    c.wait()
                e0 = (row0 + it) * 128
                pltpu.async_copy(bufD[b], gd_hbm.at[pl.ds(e0, 128)], semW[b])
                pltpu.async_copy(bufS[b], gs_hbm.at[pl.ds(e0, 128)], semW[b])
            return carry

        lax.fori_loop(0, nit // 2, pair, 0)
        wait_wb(0)
        wait_wb(1)

    return pl.kernel(body, out_type=outs, mesh=mesh, scratch_types=scratch)


def _gather(td, ts, dst2d, src2d):
    return _sc_gather_call()(td, ts, dst2d, src2d)


PASS_NODES = 12224
ACC_ROWS = 12232
MINI = 1152
N_OUT = 50048
IDX_ROWS = E_PAD // 128
ROWS_PT = IDX_ROWS // 16
SG = 1


def _sc_scatter_call():
    mesh = plsc.VectorSubcoreMesh(core_axis_name="c", subcore_axis_name="s")
    out = jax.ShapeDtypeStruct((N_OUT, 128), jnp.float32)
    scratch = [
        pltpu.VMEM((SG, 128), jnp.int32),
        pltpu.VMEM((SG, 128), jnp.int32),
        pltpu.VMEM((SG * 128, 128), jnp.float32),
        pltpu.VMEM((SG * 128, 128), jnp.float32),
        pltpu.VMEM((16,), jnp.int32),
        pltpu.VMEM_SHARED((ACC_ROWS, 128), jnp.float32),
        pltpu.SemaphoreType.DMA,
        pltpu.SemaphoreType.DMA,
        pltpu.SemaphoreType.DMA,
        pltpu.SemaphoreType.DMA,
        pltpu.SemaphoreType.DMA,
    ]

    def body(msg_hbm, lidx_hbm, zeros_hbm, bnd_hbm, agg_hbm, idxb0, idxb1,
             mbuf0, mbuf1, bref, acc, semI0, semI1, semM0, semM1, semO):
        cid = lax.axis_index("c")
        sid = lax.axis_index("s")
        idxb = (idxb0, idxb1)
        mbuf = (mbuf0, mbuf1)
        semI = (semI0, semI1)
        semM = (semM0, semM1)
        pltpu.sync_copy(bnd_hbm, bref)
        bv = bref[...]

        def getb(k):
            if k == 4:
                return jnp.int32(E_PAD)
            return bv[k]

        def zero_acc():
            @pl.when(sid < 15)
            def _():
                pltpu.async_copy(zeros_hbm, acc.at[pl.ds(sid * 768, 768)],
                                 semO).wait()

            @pl.when(sid == 15)
            def _():
                pltpu.async_copy(zeros_hbm.at[pl.ds(0, 712)],
                                 acc.at[pl.ds(11520, 712)], semO).wait()

        def stream_scatter(pidx, r0, r1):
            per = (r1 - r0 + 15) // 16
            t0 = r0 + sid * per
            nit = jnp.clip(r1 - t0, 0, per)

            def issue_loads(it, b):
                r = t0 + it
                pltpu.async_copy(
                    lidx_hbm.at[pl.ds(pidx * IDX_ROWS + r, SG)], idxb[b],
                    semI[b])
                pltpu.async_copy(
                    msg_hbm.at[pl.ds(r * 128, SG * 128)], mbuf[b], semM[b])

            def wait_loads(b):
                pltpu.make_async_copy(
                    lidx_hbm.at[pl.ds(0, SG)], idxb[b], semI[b]).wait()
                pltpu.make_async_copy(
                    msg_hbm.at[pl.ds(0, SG * 128)], mbuf[b], semM[b]).wait()

            @pl.when(nit > 0)
            def _():
                issue_loads(0, 0)

            def pair(p, carry):
                for b in range(2):
                    it = p * 2 + b

                    @pl.when(it < nit)
                    def _():
                        wait_loads(b)

                        @pl.when(it + 1 < nit)
                        def _():
                            issue_loads(it + 1, 1 - b)

                        pltpu.async_copy(
                            mbuf[b], acc.at[idxb[b].at[0]], semO,
                            add=True).wait()
                return carry

            lax.fori_loop(0, (nit + 1) // 2, pair, 0)

        for k_local in range(2):
            kidx = 2 * cid + k_local
            base = pl.multiple_of(MINI + kidx * PASS_NODES, 8)
            zero_acc()
            plsc.subcore_barrier()
            ck = jnp.where(cid == 0, getb(k_local), getb(2 + k_local))
            ck1 = jnp.where(cid == 0, getb(k_local + 1),
                            getb(2 + k_local + 1))
            stream_scatter(kidx, ck // 128, (ck1 + 127) // 128)
            plsc.subcore_barrier()

            @pl.when(sid < 15)
            def _():
                pltpu.async_copy(
                    acc.at[pl.ds(sid * 768, 768)],
                    agg_hbm.at[pl.ds(base + sid * 768, 768)], semO).wait()

            @pl.when(sid == 15)
            def _():
                pltpu.async_copy(
                    acc.at[pl.ds(11520, 704)],
                    agg_hbm.at[pl.ds(base + 11520, 704)], semO).wait()

            plsc.subcore_barrier()

        zero_acc()
        plsc.subcore_barrier()
        stream_scatter(4, 0, (getb(0) + 127) // 128)
        plsc.subcore_barrier()

        @pl.when((cid == 0) & (sid < 9))
        def _():
            pltpu.async_copy(acc.at[pl.ds(sid * 128, 128)],
                             agg_hbm.at[pl.ds(sid * 128, 128)], semO).wait()

    return pl.kernel(body, out_type=out, mesh=mesh, scratch_types=scratch)


def _scatter(msg, lidx2d, zeros784, bnd16):
    return _sc_scatter_call()(msg, lidx2d, zeros784, bnd16)


def kernel(s, v, p, edge_index_local, d_local, a_local, r_local, e_local,
           edge_index_global, d_global, a_global, r_global, e_global, batch, params):
    src = edge_index_local[0]
    dst = edge_index_local[1]
    v48 = v.reshape(N, 3 * VDIM)
    npad = E_PAD - E
    pad_ids = (jnp.arange(npad, dtype=jnp.int32) * 997) % N
    dst_seg = jnp.concatenate([dst, jnp.full((npad,), N, jnp.int32)])
    order = jnp.argsort(dst_seg)
    dst_seg = dst_seg[order]
    dst2d = jnp.concatenate([dst, pad_ids])[order].reshape(E_PAD // 128, 128)
    src2d = jnp.concatenate([src, pad_ids])[order].reshape(E_PAD // 128, 128)
    dae = jnp.concatenate([d_local[:, None], a_local[:, None], e_local], axis=-1)
    dae = jnp.concatenate([dae, jnp.zeros((npad, 18), jnp.float32)], axis=0)[order]
    r_pad = jnp.concatenate([r_local, jnp.zeros((npad, 3), jnp.float32)],
                            axis=0)[order]

    spread8 = jnp.arange(E_PAD, dtype=jnp.int32) % 8
    dump = PASS_NODES + spread8
    lidx_parts = []
    for k in range(4):
        base = MINI + k * PASS_NODES
        inr = (dst_seg >= base) & (dst_seg < base + PASS_NODES)
        lidx_parts.append(jnp.where(inr, dst_seg - base, dump))
    lidx_parts.append(jnp.where(dst_seg < MINI, dst_seg, MINI + spread8))
    lidx2d = jnp.concatenate(lidx_parts).reshape(5 * IDX_ROWS, 128)
    zeros784 = jnp.zeros((768, 128), jnp.float32)
    cuts = MINI + PASS_NODES * jnp.arange(4, dtype=jnp.int32)
    bnd16 = jnp.zeros((16,), jnp.int32).at[:4].set(
        jnp.searchsorted(dst_seg, cuts).astype(jnp.int32))

    for i in range(NL):
        lp = params["layers"][i]
        has_v = i > 0
        has_mlp = i < NL - 1
        vg48 = jnp.tile(lp["ln_vg"], 3)
        w1_dst = lp["eW1"][:SDIM]
        w1_src = lp["eW1"][SDIM:2 * SDIM]
        w1_c = lp["eW1"][2 * SDIM:]
        sln, vln, td, ts = _ln_proj(s, v48, lp["ln_g"], lp["ln_b"], vg48,
                                    w1_dst, w1_src)
        gd, gs = _gather(td, ts, dst2d, src2d)
        msg = _edge_mlp(gd, gs, dae, r_pad, w1_c, lp["eb1"],
                        lp["eW2"], lp["eb2"], has_v)
        agg = _scatter(msg, lidx2d, zeros784, bnd16)
        s, v48 = _update(sln, vln, agg,
                         lp["uW1"][:SDIM], lp["uW1"][SDIM:], lp["ub1"],
                         lp["uW2"], lp["ub2"], has_mlp)

    on = params["out_norm"]
    s, v48 = _ln_out(s, v48, on["g"], on["b"], jnp.tile(on["vg"], 3))
    return (s, v48.reshape(N, 3, VDIM))
